# Initial kernel scaffold; baseline (speedup 1.0000x reference)
import jax
import jax.numpy as jnp
from jax.experimental import pallas as pl


def kernel(inputs, weight):
    raise NotImplementedError("write your pallas kernel here")


# speedup vs baseline: 5.0312x; 5.0312x over previous
import functools

import jax
import jax.numpy as jnp
from jax.experimental import pallas as pl
from jax.experimental.pallas import tpu as pltpu

_NUM_EMB = 8192
_DIM = 256
_COMMIT = 0.25
_BN = 256


_GROUP = 2048
_BLK_PER_GRP = _GROUP // _BN


def _argmin_body(x_ref, wt_ref, x2_ref, w2_ref, idx_ref,
                 gmin_scr, gidx_scr, min_scr, idx_scr):
    j = pl.program_id(0)
    m = jax.lax.dot_general(
        x_ref[...], wt_ref[...], (((1,), (0,)), ((), ())),
        preferred_element_type=jnp.float32)
    dist = (x2_ref[...] + w2_ref[...]) - 2.0 * m
    bmin = jnp.min(dist, axis=1, keepdims=True)
    iota = jax.lax.broadcasted_iota(jnp.int32, dist.shape, 1)
    bidx = jnp.min(jnp.where(dist == bmin, iota, _NUM_EMB),
                   axis=1, keepdims=True) + j * _BN

    @pl.when(j % _BLK_PER_GRP == 0)
    def _():
        gmin_scr[...] = bmin
        gidx_scr[...] = bidx

    @pl.when(j % _BLK_PER_GRP != 0)
    def _():
        better = bmin < gmin_scr[...]
        gmin_scr[...] = jnp.where(better, bmin, gmin_scr[...])
        gidx_scr[...] = jnp.where(better, bidx, gidx_scr[...])

    @pl.when(j == _BLK_PER_GRP - 1)
    def _():
        min_scr[...] = gmin_scr[...].astype(jnp.bfloat16).astype(jnp.float32)
        idx_scr[...] = gidx_scr[...]

    @pl.when(jnp.logical_and(j % _BLK_PER_GRP == _BLK_PER_GRP - 1,
                             j >= _BLK_PER_GRP))
    def _():
        upd = gmin_scr[...] < min_scr[...]
        min_scr[...] = jnp.where(
            upd, gmin_scr[...].astype(jnp.bfloat16).astype(jnp.float32),
            min_scr[...])
        idx_scr[...] = jnp.where(upd, gidx_scr[...], idx_scr[...])

    @pl.when(j == pl.num_programs(0) - 1)
    def _():
        idx_ref[...] = idx_scr[...]


@functools.partial(jax.jit, static_argnames=("interpret",))
def _nearest_indices(flat, wt, x2, w2, interpret=False):
    n = flat.shape[0]
    grid = _NUM_EMB // _BN
    return pl.pallas_call(
        _argmin_body,
        grid=(grid,),
        in_specs=[
            pl.BlockSpec((n, _DIM), lambda j: (0, 0)),
            pl.BlockSpec((_DIM, _BN), lambda j: (0, j)),
            pl.BlockSpec((n, 1), lambda j: (0, 0)),
            pl.BlockSpec((1, _BN), lambda j: (0, j)),
        ],
        out_specs=pl.BlockSpec((n, 1), lambda j: (0, 0)),
        out_shape=jax.ShapeDtypeStruct((n, 1), jnp.int32),
        scratch_shapes=[
            pltpu.VMEM((n, 1), jnp.float32),
            pltpu.VMEM((n, 1), jnp.int32),
            pltpu.VMEM((n, 1), jnp.float32),
            pltpu.VMEM((n, 1), jnp.int32),
        ],
        interpret=interpret,
    )(flat, wt, x2, w2)


def kernel(inputs, weight, interpret=False):
    input_shape = inputs.shape
    flat = inputs.reshape(-1, _DIM)
    n = flat.shape[0]
    x2 = jnp.sum(flat ** 2, axis=1, keepdims=True)
    w2 = jnp.sum(weight ** 2, axis=1)[None, :]
    idx = _nearest_indices(flat, weight.T, x2, w2, interpret=interpret)
    idx_flat = idx[:, 0]
    quantized = weight[idx_flat].reshape(input_shape)
    e_latent = jnp.mean((quantized - inputs) ** 2)
    q_latent = jnp.mean((quantized - inputs) ** 2)
    loss = q_latent + _COMMIT * e_latent
    quantized_st = inputs + (quantized - inputs)
    counts = jnp.zeros((_NUM_EMB,), jnp.float32).at[idx_flat].add(1.0)
    avg_probs = counts / n
    perplexity = jnp.exp(-jnp.sum(avg_probs * jnp.log(avg_probs + 1e-10)))
    return (loss, quantized_st, perplexity, idx_flat.reshape(1, -1))


# speedup vs baseline: 5.7018x; 1.1333x over previous
import functools

import jax
import jax.numpy as jnp
from jax import lax
from jax.experimental import pallas as pl
from jax.experimental.pallas import tpu as pltpu
from jax.experimental.pallas import tpu_sc as plsc

_NUM_EMB = 8192
_DIM = 256
_COMMIT = 0.25
_BN = 256


_GROUP = 2048
_BLK_PER_GRP = _GROUP // _BN


def _argmin_body(x_ref, wt_ref, x2_ref, w2_ref, idx_ref,
                 gmin_scr, gidx_scr, min_scr, idx_scr):
    j = pl.program_id(0)
    m = jax.lax.dot_general(
        x_ref[...], wt_ref[...], (((1,), (0,)), ((), ())),
        preferred_element_type=jnp.float32)
    dist = (x2_ref[...] + w2_ref[...]) - 2.0 * m
    bmin = jnp.min(dist, axis=1, keepdims=True)
    iota = jax.lax.broadcasted_iota(jnp.int32, dist.shape, 1)
    bidx = jnp.min(jnp.where(dist == bmin, iota, _NUM_EMB),
                   axis=1, keepdims=True) + j * _BN

    @pl.when(j % _BLK_PER_GRP == 0)
    def _():
        gmin_scr[...] = bmin
        gidx_scr[...] = bidx

    @pl.when(j % _BLK_PER_GRP != 0)
    def _():
        better = bmin < gmin_scr[...]
        gmin_scr[...] = jnp.where(better, bmin, gmin_scr[...])
        gidx_scr[...] = jnp.where(better, bidx, gidx_scr[...])

    @pl.when(j == _BLK_PER_GRP - 1)
    def _():
        min_scr[...] = gmin_scr[...].astype(jnp.bfloat16).astype(jnp.float32)
        idx_scr[...] = gidx_scr[...]

    @pl.when(jnp.logical_and(j % _BLK_PER_GRP == _BLK_PER_GRP - 1,
                             j >= _BLK_PER_GRP))
    def _():
        upd = gmin_scr[...] < min_scr[...]
        min_scr[...] = jnp.where(
            upd, gmin_scr[...].astype(jnp.bfloat16).astype(jnp.float32),
            min_scr[...])
        idx_scr[...] = jnp.where(upd, gidx_scr[...], idx_scr[...])

    @pl.when(j == pl.num_programs(0) - 1)
    def _():
        idx_ref[...] = idx_scr[...]


@jax.jit
def _nearest_indices(flat, wt, x2, w2):
    n = flat.shape[0]
    grid = _NUM_EMB // _BN
    return pl.pallas_call(
        _argmin_body,
        grid=(grid,),
        in_specs=[
            pl.BlockSpec((n, _DIM), lambda j: (0, 0)),
            pl.BlockSpec((_DIM, _BN), lambda j: (0, j)),
            pl.BlockSpec((n, 1), lambda j: (0, 0)),
            pl.BlockSpec((1, _BN), lambda j: (0, j)),
        ],
        out_specs=pl.BlockSpec((n, 1), lambda j: (0, 0)),
        out_shape=jax.ShapeDtypeStruct((n, 1), jnp.int32),
        scratch_shapes=[
            pltpu.VMEM((n, 1), jnp.float32),
            pltpu.VMEM((n, 1), jnp.int32),
            pltpu.VMEM((n, 1), jnp.float32),
            pltpu.VMEM((n, 1), jnp.int32),
        ],
    )(flat, wt, x2, w2)


_CNT_W = 16


def _sc_gather_counts(weight, idx):
    info = plsc.get_sparse_core_info()
    nc, ns = info.num_cores, info.num_subcores
    nw = nc * ns
    b = idx.shape[0]
    d = weight.shape[1]
    bpw = b // nw
    mesh = plsc.VectorSubcoreMesh(core_axis_name="c", subcore_axis_name="s")

    @functools.partial(
        pl.kernel, mesh=mesh,
        out_type=jax.ShapeDtypeStruct((b, d), jnp.float32),
        scratch_types=[
            pltpu.VMEM((bpw,), jnp.int32),
            pltpu.VMEM((bpw, d), jnp.float32),
            pltpu.SemaphoreType.DMA,
        ],
    )
    def k(weight_hbm, idx_hbm, out_hbm, idx_v, rows_v, sem):
        wid = lax.axis_index("s") * nc + lax.axis_index("c")
        base = wid * bpw
        pltpu.sync_copy(idx_hbm.at[pl.ds(base, bpw)], idx_v)
        pltpu.async_copy(weight_hbm.at[idx_v], rows_v, sem).wait()
        pltpu.sync_copy(rows_v, out_hbm.at[pl.ds(base, bpw)])

    return k(weight, idx)


def _count_body(idx_ref, cnt_ref):
    j = pl.program_id(0)
    codes = jax.lax.broadcasted_iota(jnp.int32, (8192, 512), 1) + j * 512
    eq = (idx_ref[...] == codes).astype(jnp.float32)
    cnt_ref[...] = jnp.sum(eq, axis=0, keepdims=True)


@jax.jit
def _code_counts(idx_col):
    return pl.pallas_call(
        _count_body,
        grid=(_NUM_EMB // 512,),
        in_specs=[pl.BlockSpec((8192, 1), lambda j: (0, 0))],
        out_specs=pl.BlockSpec((1, 512), lambda j: (0, j)),
        out_shape=jax.ShapeDtypeStruct((1, _NUM_EMB), jnp.float32),
    )(idx_col)


def _finish_body(x_ref, q_ref, cnt_ref, qst_ref, loss_ref, perp_ref):
    x = x_ref[...]
    q = q_ref[...]
    qst_ref[...] = x + (q - x)
    diff2 = (q - x) * (q - x)
    e_latent = jnp.sum(diff2, keepdims=True) * (1.0 / (8192.0 * 256.0))
    loss_ref[...] = (e_latent + _COMMIT * e_latent).reshape(1, 1)
    avg = cnt_ref[...] * (1.0 / 8192.0)
    ent = jnp.sum(avg * jnp.log(avg + 1e-10), keepdims=True)
    perp_ref[...] = jnp.exp(-ent).reshape(1, 1)


@jax.jit
def _finish(flat, quant, counts_row):
    return pl.pallas_call(
        _finish_body,
        in_specs=[pl.BlockSpec(flat.shape, lambda: (0, 0)),
                  pl.BlockSpec(flat.shape, lambda: (0, 0)),
                  pl.BlockSpec(counts_row.shape, lambda: (0, 0))],
        out_specs=[pl.BlockSpec(flat.shape, lambda: (0, 0)),
                   pl.BlockSpec((1, 1), lambda: (0, 0)),
                   pl.BlockSpec((1, 1), lambda: (0, 0))],
        out_shape=[jax.ShapeDtypeStruct(flat.shape, jnp.float32),
                   jax.ShapeDtypeStruct((1, 1), jnp.float32),
                   jax.ShapeDtypeStruct((1, 1), jnp.float32)],
    )(flat, quant, counts_row)


def kernel(inputs, weight):
    input_shape = inputs.shape
    flat = inputs.reshape(-1, _DIM)
    x2 = jnp.sum(flat ** 2, axis=1, keepdims=True)
    w2 = jnp.sum(weight ** 2, axis=1)[None, :]
    idx = _nearest_indices(flat, weight.T, x2, w2)
    idx_flat = idx[:, 0]
    quant = _sc_gather_counts(weight, idx_flat)
    counts = _code_counts(idx)
    qst, loss, perp = _finish(flat, quant, counts)
    return (loss[0, 0], qst.reshape(input_shape), perp[0, 0],
            idx_flat.reshape(1, -1))


# speedup vs baseline: 5.8337x; 1.0231x over previous
import functools

import jax
import jax.numpy as jnp
from jax import lax
from jax.experimental import pallas as pl
from jax.experimental.pallas import tpu as pltpu
from jax.experimental.pallas import tpu_sc as plsc

_NUM_EMB = 8192
_DIM = 256
_COMMIT = 0.25
_BN = 256


_GROUP = 2048
_BLK_PER_GRP = _GROUP // _BN


def _argmin_body(x_ref, wt_ref, x2_ref, w2_ref, idx_ref,
                 gmin_scr, gidx_scr, min_scr, idx_scr):
    j = pl.program_id(0)
    m = jax.lax.dot_general(
        x_ref[...], wt_ref[...], (((1,), (1,)), ((), ())),
        preferred_element_type=jnp.float32)
    dist = (x2_ref[...] + w2_ref[...]) - 2.0 * m
    bmin = jnp.min(dist, axis=1, keepdims=True)
    iota = jax.lax.broadcasted_iota(jnp.int32, dist.shape, 1)
    bidx = jnp.min(jnp.where(dist == bmin, iota, _NUM_EMB),
                   axis=1, keepdims=True) + j * _BN

    @pl.when(j % _BLK_PER_GRP == 0)
    def _():
        gmin_scr[...] = bmin
        gidx_scr[...] = bidx

    @pl.when(j % _BLK_PER_GRP != 0)
    def _():
        better = bmin < gmin_scr[...]
        gmin_scr[...] = jnp.where(better, bmin, gmin_scr[...])
        gidx_scr[...] = jnp.where(better, bidx, gidx_scr[...])

    @pl.when(j == _BLK_PER_GRP - 1)
    def _():
        min_scr[...] = gmin_scr[...].astype(jnp.bfloat16).astype(jnp.float32)
        idx_scr[...] = gidx_scr[...]

    @pl.when(jnp.logical_and(j % _BLK_PER_GRP == _BLK_PER_GRP - 1,
                             j >= _BLK_PER_GRP))
    def _():
        upd = gmin_scr[...] < min_scr[...]
        min_scr[...] = jnp.where(
            upd, gmin_scr[...].astype(jnp.bfloat16).astype(jnp.float32),
            min_scr[...])
        idx_scr[...] = jnp.where(upd, gidx_scr[...], idx_scr[...])

    @pl.when(j == pl.num_programs(0) - 1)
    def _():
        idx_ref[...] = idx_scr[...]


@jax.jit
def _nearest_indices(flat, wt, x2, w2):
    n = flat.shape[0]
    grid = _NUM_EMB // _BN
    return pl.pallas_call(
        _argmin_body,
        grid=(grid,),
        in_specs=[
            pl.BlockSpec((n, _DIM), lambda j: (0, 0)),
            pl.BlockSpec((_BN, _DIM), lambda j: (j, 0)),
            pl.BlockSpec((n, 1), lambda j: (0, 0)),
            pl.BlockSpec((1, _BN), lambda j: (0, j)),
        ],
        out_specs=pl.BlockSpec((n, 1), lambda j: (0, 0)),
        out_shape=jax.ShapeDtypeStruct((n, 1), jnp.int32),
        scratch_shapes=[
            pltpu.VMEM((n, 1), jnp.float32),
            pltpu.VMEM((n, 1), jnp.int32),
            pltpu.VMEM((n, 1), jnp.float32),
            pltpu.VMEM((n, 1), jnp.int32),
        ],
    )(flat, wt, x2, w2)


_CNT_W = 16


def _sc_gather_counts(weight, idx):
    info = plsc.get_sparse_core_info()
    nc, ns = info.num_cores, info.num_subcores
    nw = nc * ns
    b = idx.shape[0]
    d = weight.shape[1]
    bpw = b // nw
    mesh = plsc.VectorSubcoreMesh(core_axis_name="c", subcore_axis_name="s")

    @functools.partial(
        pl.kernel, mesh=mesh,
        out_type=jax.ShapeDtypeStruct((b, d), jnp.float32),
        scratch_types=[
            pltpu.VMEM((bpw,), jnp.int32),
            pltpu.VMEM((bpw, d), jnp.float32),
            pltpu.SemaphoreType.DMA,
        ],
    )
    def k(weight_hbm, idx_hbm, out_hbm, idx_v, rows_v, sem):
        wid = lax.axis_index("s") * nc + lax.axis_index("c")
        base = wid * bpw
        pltpu.sync_copy(idx_hbm.at[pl.ds(base, bpw)], idx_v)
        pltpu.async_copy(weight_hbm.at[idx_v], rows_v, sem).wait()
        pltpu.sync_copy(rows_v, out_hbm.at[pl.ds(base, bpw)])

    return k(weight, idx)


def _count_body(idx_ref, cnt_ref):
    j = pl.program_id(0)
    codes = jax.lax.broadcasted_iota(jnp.int32, (8192, 512), 1) + j * 512
    eq = (idx_ref[...] == codes).astype(jnp.float32)
    cnt_ref[...] = jnp.sum(eq, axis=0, keepdims=True)


@jax.jit
def _code_counts(idx_col):
    return pl.pallas_call(
        _count_body,
        grid=(_NUM_EMB // 512,),
        in_specs=[pl.BlockSpec((8192, 1), lambda j: (0, 0))],
        out_specs=pl.BlockSpec((1, 512), lambda j: (0, j)),
        out_shape=jax.ShapeDtypeStruct((1, _NUM_EMB), jnp.float32),
    )(idx_col)


def _finish_body(x_ref, q_ref, cnt_ref, qst_ref, loss_ref, perp_ref):
    x = x_ref[...]
    q = q_ref[...]
    qst_ref[...] = x + (q - x)
    diff2 = (q - x) * (q - x)
    e_latent = jnp.sum(diff2, keepdims=True) * (1.0 / (8192.0 * 256.0))
    loss_ref[...] = (e_latent + _COMMIT * e_latent).reshape(1, 1)
    avg = cnt_ref[...] * (1.0 / 8192.0)
    ent = jnp.sum(avg * jnp.log(avg + 1e-10), keepdims=True)
    perp_ref[...] = jnp.exp(-ent).reshape(1, 1)


@jax.jit
def _finish(flat, quant, counts_row):
    return pl.pallas_call(
        _finish_body,
        in_specs=[pl.BlockSpec(flat.shape, lambda: (0, 0)),
                  pl.BlockSpec(flat.shape, lambda: (0, 0)),
                  pl.BlockSpec(counts_row.shape, lambda: (0, 0))],
        out_specs=[pl.BlockSpec(flat.shape, lambda: (0, 0)),
                   pl.BlockSpec((1, 1), lambda: (0, 0)),
                   pl.BlockSpec((1, 1), lambda: (0, 0))],
        out_shape=[jax.ShapeDtypeStruct(flat.shape, jnp.float32),
                   jax.ShapeDtypeStruct((1, 1), jnp.float32),
                   jax.ShapeDtypeStruct((1, 1), jnp.float32)],
    )(flat, quant, counts_row)


def kernel(inputs, weight):
    input_shape = inputs.shape
    flat = inputs.reshape(-1, _DIM)
    x2 = jnp.sum(flat ** 2, axis=1, keepdims=True)
    w2 = jnp.sum(weight ** 2, axis=1)[None, :]
    idx = _nearest_indices(flat, weight, x2, w2)
    idx_flat = idx[:, 0]
    quant = _sc_gather_counts(weight, idx_flat)
    counts = _code_counts(idx)
    qst, loss, perp = _finish(flat, quant, counts)
    return (loss[0, 0], qst.reshape(input_shape), perp[0, 0],
            idx_flat.reshape(1, -1))


# speedup vs baseline: 5.8425x; 1.0015x over previous
import functools

import jax
import jax.numpy as jnp
from jax import lax
from jax.experimental import pallas as pl
from jax.experimental.pallas import tpu as pltpu
from jax.experimental.pallas import tpu_sc as plsc

_NUM_EMB = 8192
_DIM = 256
_COMMIT = 0.25
_BN = 256


_GROUP = 2048
_BLK_PER_GRP = _GROUP // _BN


def _argmin_body(x_ref, wt_ref, x2_ref, w2_ref, idx_ref,
                 gmin_scr, gidx_scr, min_scr, idx_scr):
    j = pl.program_id(0)
    m = jax.lax.dot_general(
        x_ref[...], wt_ref[...], (((1,), (1,)), ((), ())),
        preferred_element_type=jnp.float32)
    dist = (x2_ref[...] + w2_ref[...]) - 2.0 * m
    bmin = jnp.min(dist, axis=1, keepdims=True)
    iota = jax.lax.broadcasted_iota(jnp.int32, dist.shape, 1)
    bidx = jnp.min(jnp.where(dist == bmin, iota, _NUM_EMB),
                   axis=1, keepdims=True) + j * _BN

    @pl.when(j % _BLK_PER_GRP == 0)
    def _():
        gmin_scr[...] = bmin
        gidx_scr[...] = bidx

    @pl.when(j % _BLK_PER_GRP != 0)
    def _():
        better = bmin < gmin_scr[...]
        gmin_scr[...] = jnp.where(better, bmin, gmin_scr[...])
        gidx_scr[...] = jnp.where(better, bidx, gidx_scr[...])

    @pl.when(j == _BLK_PER_GRP - 1)
    def _():
        min_scr[...] = gmin_scr[...].astype(jnp.bfloat16).astype(jnp.float32)
        idx_scr[...] = gidx_scr[...]

    @pl.when(jnp.logical_and(j % _BLK_PER_GRP == _BLK_PER_GRP - 1,
                             j >= _BLK_PER_GRP))
    def _():
        upd = gmin_scr[...] < min_scr[...]
        min_scr[...] = jnp.where(
            upd, gmin_scr[...].astype(jnp.bfloat16).astype(jnp.float32),
            min_scr[...])
        idx_scr[...] = jnp.where(upd, gidx_scr[...], idx_scr[...])

    @pl.when(j == pl.num_programs(0) - 1)
    def _():
        idx_ref[...] = idx_scr[...]


@jax.jit
def _nearest_indices(flat, wt, x2, w2):
    n = flat.shape[0]
    grid = _NUM_EMB // _BN
    return pl.pallas_call(
        _argmin_body,
        grid=(grid,),
        in_specs=[
            pl.BlockSpec((n, _DIM), lambda j: (0, 0)),
            pl.BlockSpec((_BN, _DIM), lambda j: (j, 0)),
            pl.BlockSpec((n, 1), lambda j: (0, 0)),
            pl.BlockSpec((1, _BN), lambda j: (0, j)),
        ],
        out_specs=pl.BlockSpec((n, 1), lambda j: (0, 0)),
        out_shape=jax.ShapeDtypeStruct((n, 1), jnp.int32),
        scratch_shapes=[
            pltpu.VMEM((n, 1), jnp.float32),
            pltpu.VMEM((n, 1), jnp.int32),
            pltpu.VMEM((n, 1), jnp.float32),
            pltpu.VMEM((n, 1), jnp.int32),
        ],
    )(flat, wt, x2, w2)


def _sc_gather(weight, idx):
    info = plsc.get_sparse_core_info()
    nc, ns = info.num_cores, info.num_subcores
    nw = nc * ns
    b = idx.shape[0]
    d = weight.shape[1]
    bpw = b // nw
    mesh = plsc.VectorSubcoreMesh(core_axis_name="c", subcore_axis_name="s")

    @functools.partial(
        pl.kernel, mesh=mesh,
        out_type=jax.ShapeDtypeStruct((b, d), jnp.float32),
        scratch_types=[
            pltpu.VMEM((bpw,), jnp.int32),
            pltpu.VMEM((bpw, d), jnp.float32),
            pltpu.SemaphoreType.DMA,
        ],
    )
    def k(weight_hbm, idx_hbm, out_hbm, idx_v, rows_v, sem):
        wid = lax.axis_index("s") * nc + lax.axis_index("c")
        base = wid * bpw
        pltpu.sync_copy(idx_hbm.at[pl.ds(base, bpw)], idx_v)
        pltpu.async_copy(weight_hbm.at[idx_v], rows_v, sem).wait()
        pltpu.sync_copy(rows_v, out_hbm.at[pl.ds(base, bpw)])

    return k(weight, idx)


def _count_body(idx_ref, cnt_ref):
    j = pl.program_id(0)
    codes = jax.lax.broadcasted_iota(jnp.int32, (8192, 512), 1) + j * 512
    eq = (idx_ref[...] == codes).astype(jnp.float32)
    cnt_ref[...] = jnp.sum(eq, axis=0, keepdims=True)


@jax.jit
def _code_counts(idx_col):
    return pl.pallas_call(
        _count_body,
        grid=(_NUM_EMB // 512,),
        in_specs=[pl.BlockSpec((8192, 1), lambda j: (0, 0))],
        out_specs=pl.BlockSpec((1, 512), lambda j: (0, j)),
        out_shape=jax.ShapeDtypeStruct((1, _NUM_EMB), jnp.float32),
    )(idx_col)


def _finish_body(x_ref, q_ref, cnt_ref, qst_ref, loss_ref, perp_ref):
    x = x_ref[...]
    q = q_ref[...]
    qst_ref[...] = x + (q - x)
    diff2 = (q - x) * (q - x)
    e_latent = jnp.sum(diff2, keepdims=True) * (1.0 / (8192.0 * 256.0))
    loss_ref[...] = (e_latent + _COMMIT * e_latent).reshape(1, 1)
    avg = cnt_ref[...] * (1.0 / 8192.0)
    ent = jnp.sum(avg * jnp.log(avg + 1e-10), keepdims=True)
    perp_ref[...] = jnp.exp(-ent).reshape(1, 1)


@jax.jit
def _finish(flat, quant, counts_row):
    return pl.pallas_call(
        _finish_body,
        in_specs=[pl.BlockSpec(flat.shape, lambda: (0, 0)),
                  pl.BlockSpec(flat.shape, lambda: (0, 0)),
                  pl.BlockSpec(counts_row.shape, lambda: (0, 0))],
        out_specs=[pl.BlockSpec(flat.shape, lambda: (0, 0)),
                   pl.BlockSpec((1, 1), lambda: (0, 0)),
                   pl.BlockSpec((1, 1), lambda: (0, 0))],
        out_shape=[jax.ShapeDtypeStruct(flat.shape, jnp.float32),
                   jax.ShapeDtypeStruct((1, 1), jnp.float32),
                   jax.ShapeDtypeStruct((1, 1), jnp.float32)],
    )(flat, quant, counts_row)


def kernel(inputs, weight):
    input_shape = inputs.shape
    flat = inputs.reshape(-1, _DIM)
    x2 = jnp.sum(flat ** 2, axis=1, keepdims=True)
    w2 = jnp.sum(weight ** 2, axis=1)[None, :]
    idx = _nearest_indices(flat, weight, x2, w2)
    idx_flat = idx[:, 0]
    quant = _sc_gather(weight, idx_flat)
    counts = _code_counts(idx)
    qst, loss, perp = _finish(flat, quant, counts)
    return (loss[0, 0], qst.reshape(input_shape), perp[0, 0],
            idx_flat.reshape(1, -1))
